# Initial kernel scaffold; baseline (speedup 1.0000x reference)
import jax
import jax.numpy as jnp
from jax.experimental import pallas as pl


def kernel(soft_masks, visual_features, class_logits):
    raise NotImplementedError("write your pallas kernel here")


# speedup vs baseline: 1.0536x; 1.0536x over previous
import functools

import jax
import jax.numpy as jnp
from jax import lax
from jax.experimental import pallas as pl
from jax.experimental.pallas import tpu as pltpu

B, Q, HM, WM = 4, 200, 128, 128
T, D = 1024, 768
C1 = 81
TOP_K = 16
EPS = 1e-06
GH = 32


def _scores_topk_body(logits_ref, scores_ref, tks_ref, tki_ref, tkc_ref):
    l = logits_ref[0]
    m = jnp.max(l, axis=-1, keepdims=True)
    p = jnp.exp(l - m)
    s = jnp.sum(p, axis=-1, keepdims=True)
    probs = p / s
    fg = probs[:, : C1 - 1]
    sc = jnp.max(fg, axis=-1, keepdims=True)
    sc_row = sc.T
    scores_ref[0] = sc_row

    cio = lax.broadcasted_iota(jnp.int32, (Q, C1 - 1), 1)
    cls = jnp.min(jnp.where(fg == sc, cio, C1), axis=-1, keepdims=True)

    ii = lax.broadcasted_iota(jnp.int32, (Q, Q), 0)
    jj = lax.broadcasted_iota(jnp.int32, (Q, Q), 1)
    beats = (sc_row > sc) | ((sc_row == sc) & (jj < ii))
    rank = jnp.sum(beats.astype(jnp.int32), axis=-1, keepdims=True)

    ro = lax.broadcasted_iota(jnp.int32, (Q, TOP_K), 1)
    onehot = rank == ro
    qio = lax.broadcasted_iota(jnp.int32, (Q, TOP_K), 0)
    tks_ref[0] = jnp.sum(jnp.where(onehot, sc, 0.0), axis=0, keepdims=True)
    tki_ref[0] = jnp.sum(jnp.where(onehot, qio, 0), axis=0, keepdims=True)
    tkc_ref[0] = jnp.sum(jnp.where(onehot, cls, 0), axis=0, keepdims=True)


def _pool_body(idx_ref, mask_ref, v_ref, out_ref, w_scratch):
    i = pl.program_id(1)

    m = mask_ref[0, 0]
    r = lax.broadcasted_iota(jnp.int32, (GH, HM), 0)
    c = lax.broadcasted_iota(jnp.int32, (GH, HM), 1)
    sel = ((c == 4 * r + 1) | (c == 4 * r + 2)).astype(jnp.float32) * 0.5
    rT = lax.broadcasted_iota(jnp.int32, (WM, GH), 0)
    cT = lax.broadcasted_iota(jnp.int32, (WM, GH), 1)
    selT = ((rT == 4 * cT + 1) | (rT == 4 * cT + 2)).astype(jnp.float32) * 0.5

    hi = lax.Precision.HIGHEST
    resized = jnp.dot(
        jnp.dot(sel, m, precision=hi, preferred_element_type=jnp.float32),
        selT,
        precision=hi,
        preferred_element_type=jnp.float32,
    )
    w_scratch[:, pl.ds(i, 1), :] = resized[:, None, :]

    @pl.when(i == TOP_K - 1)
    def _():
        num = jnp.zeros((TOP_K, D), jnp.float32)
        den = jnp.zeros((TOP_K, 1), jnp.float32)
        for g in range(GH):
            sg = w_scratch[g]
            vg = v_ref[0, pl.ds(g * GH, GH), :]
            num = num + jnp.dot(sg, vg, precision=lax.Precision.HIGHEST,
                                preferred_element_type=jnp.float32)
            den = den + jnp.sum(sg, axis=1, keepdims=True)
        out_ref[0] = num / (den + EPS)


def _scores_topk(class_logits):
    return pl.pallas_call(
        _scores_topk_body,
        grid=(B,),
        in_specs=[pl.BlockSpec((1, Q, C1), lambda b: (b, 0, 0))],
        out_specs=(
            pl.BlockSpec((1, 1, Q), lambda b: (b, 0, 0)),
            pl.BlockSpec((1, 1, TOP_K), lambda b: (b, 0, 0)),
            pl.BlockSpec((1, 1, TOP_K), lambda b: (b, 0, 0)),
            pl.BlockSpec((1, 1, TOP_K), lambda b: (b, 0, 0)),
        ),
        out_shape=(
            jax.ShapeDtypeStruct((B, 1, Q), jnp.float32),
            jax.ShapeDtypeStruct((B, 1, TOP_K), jnp.float32),
            jax.ShapeDtypeStruct((B, 1, TOP_K), jnp.int32),
            jax.ShapeDtypeStruct((B, 1, TOP_K), jnp.int32),
        ),
    )(class_logits)


def _pool(topk_indices, soft_masks, visual_features):
    grid_spec = pltpu.PrefetchScalarGridSpec(
        num_scalar_prefetch=1,
        grid=(B, TOP_K),
        in_specs=[
            pl.BlockSpec((1, 1, HM, WM), lambda b, i, idx: (b, idx[b, i], 0, 0)),
            pl.BlockSpec((1, T, D), lambda b, i, idx: (b, 0, 0)),
        ],
        out_specs=pl.BlockSpec((1, TOP_K, D), lambda b, i, idx: (b, 0, 0)),
        scratch_shapes=[
            pltpu.VMEM((GH, TOP_K, GH), jnp.float32),
        ],
    )
    return pl.pallas_call(
        _pool_body,
        grid_spec=grid_spec,
        out_shape=jax.ShapeDtypeStruct((B, TOP_K, D), jnp.float32),
    )(topk_indices, soft_masks, visual_features)


@functools.partial(jax.jit)
def kernel(soft_masks, visual_features, class_logits):
    scores, topk_scores, topk_indices, topk_class_ids = _scores_topk(class_logits)
    scores = scores.reshape(B, Q)
    topk_scores = topk_scores.reshape(B, TOP_K)
    topk_indices = topk_indices.reshape(B, TOP_K)
    topk_class_ids = topk_class_ids.reshape(B, TOP_K)
    pooled = _pool(topk_indices, soft_masks, visual_features)
    return pooled, topk_scores, topk_indices, topk_class_ids, scores


# speedup vs baseline: 1.3185x; 1.2514x over previous
import functools

import jax
import jax.numpy as jnp
from jax import lax
from jax.experimental import pallas as pl
from jax.experimental.pallas import tpu as pltpu

B, Q, HM, WM = 4, 200, 128, 128
T, D = 1024, 768
C1 = 81
TOP_K = 16
EPS = 1e-06
GH = 32


def _scores_topk_body(logits_ref, scores_ref, tks_ref, tki_ref, tkc_ref):
    l = logits_ref[0]
    m = jnp.max(l, axis=-1, keepdims=True)
    p = jnp.exp(l - m)
    s = jnp.sum(p, axis=-1, keepdims=True)
    probs = p / s
    fg = probs[:, : C1 - 1]
    sc = jnp.max(fg, axis=-1, keepdims=True)
    sc_row = sc.T
    scores_ref[0] = sc_row

    cio = lax.broadcasted_iota(jnp.int32, (Q, C1 - 1), 1)
    cls = jnp.min(jnp.where(fg == sc, cio, C1), axis=-1, keepdims=True)

    ii = lax.broadcasted_iota(jnp.int32, (Q, Q), 0)
    jj = lax.broadcasted_iota(jnp.int32, (Q, Q), 1)
    beats = (sc_row > sc) | ((sc_row == sc) & (jj < ii))
    rank = jnp.sum(beats.astype(jnp.int32), axis=-1, keepdims=True)

    ro = lax.broadcasted_iota(jnp.int32, (Q, TOP_K), 1)
    onehot = rank == ro
    qio = lax.broadcasted_iota(jnp.int32, (Q, TOP_K), 0)
    tks_ref[0] = jnp.sum(jnp.where(onehot, sc, 0.0), axis=0, keepdims=True)
    tki_ref[0] = jnp.sum(jnp.where(onehot, qio, 0), axis=0, keepdims=True)
    tkc_ref[0] = jnp.sum(jnp.where(onehot, cls, 0), axis=0, keepdims=True)


def _pool_body(idx_ref, mask_ref, v_ref, out_ref, w_scratch):
    i = pl.program_id(1)

    m = mask_ref[0, 0]
    m4 = m.reshape(GH, 4, WM)
    rowsum = m4[:, 1, :] + m4[:, 2, :]
    rt4 = rowsum.T.reshape(GH, 4, GH)
    resized = (0.25 * (rt4[:, 1, :] + rt4[:, 2, :])).T
    w_scratch[:, pl.ds(i, 1), :] = resized[:, None, :]

    @pl.when(i == TOP_K - 1)
    def _():
        num = jnp.zeros((TOP_K, D), jnp.float32)
        den = jnp.zeros((TOP_K, 1), jnp.float32)
        for g in range(GH):
            sg = w_scratch[g]
            vg = v_ref[0, pl.ds(g * GH, GH), :]
            num = num + jnp.dot(sg, vg, preferred_element_type=jnp.float32)
            den = den + jnp.sum(sg, axis=1, keepdims=True)
        out_ref[0] = num / (den + EPS)


def _scores_topk(class_logits):
    return pl.pallas_call(
        _scores_topk_body,
        grid=(B,),
        in_specs=[pl.BlockSpec((1, Q, C1), lambda b: (b, 0, 0))],
        out_specs=(
            pl.BlockSpec((1, 1, Q), lambda b: (b, 0, 0)),
            pl.BlockSpec((1, 1, TOP_K), lambda b: (b, 0, 0)),
            pl.BlockSpec((1, 1, TOP_K), lambda b: (b, 0, 0)),
            pl.BlockSpec((1, 1, TOP_K), lambda b: (b, 0, 0)),
        ),
        out_shape=(
            jax.ShapeDtypeStruct((B, 1, Q), jnp.float32),
            jax.ShapeDtypeStruct((B, 1, TOP_K), jnp.float32),
            jax.ShapeDtypeStruct((B, 1, TOP_K), jnp.int32),
            jax.ShapeDtypeStruct((B, 1, TOP_K), jnp.int32),
        ),
    )(class_logits)


def _pool(topk_indices, soft_masks, visual_features):
    grid_spec = pltpu.PrefetchScalarGridSpec(
        num_scalar_prefetch=1,
        grid=(B, TOP_K),
        in_specs=[
            pl.BlockSpec((1, 1, HM, WM), lambda b, i, idx: (b, idx[b, i], 0, 0)),
            pl.BlockSpec((1, T, D), lambda b, i, idx: (b, 0, 0)),
        ],
        out_specs=pl.BlockSpec((1, TOP_K, D), lambda b, i, idx: (b, 0, 0)),
        scratch_shapes=[
            pltpu.VMEM((GH, TOP_K, GH), jnp.float32),
        ],
    )
    return pl.pallas_call(
        _pool_body,
        grid_spec=grid_spec,
        out_shape=jax.ShapeDtypeStruct((B, TOP_K, D), jnp.float32),
    )(topk_indices, soft_masks, visual_features)


@functools.partial(jax.jit)
def kernel(soft_masks, visual_features, class_logits):
    scores, topk_scores, topk_indices, topk_class_ids = _scores_topk(class_logits)
    scores = scores.reshape(B, Q)
    topk_scores = topk_scores.reshape(B, TOP_K)
    topk_indices = topk_indices.reshape(B, TOP_K)
    topk_class_ids = topk_class_ids.reshape(B, TOP_K)
    pooled = _pool(topk_indices, soft_masks, visual_features)
    return pooled, topk_scores, topk_indices, topk_class_ids, scores


# speedup vs baseline: 3.5453x; 2.6889x over previous
import functools

import jax
import jax.numpy as jnp
from jax import lax
from jax.experimental import pallas as pl
from jax.experimental.pallas import tpu as pltpu

B, Q, HM, WM = 4, 200, 128, 128
T, D = 1024, 768
C1 = 81
TOP_K = 16
EPS = 1e-06
GH = 32


def _scores_topk_body(logits_ref, scores_ref, tks_ref, tki_ref, tkc_ref):
    l = logits_ref[0]
    m = jnp.max(l, axis=-1, keepdims=True)
    p = jnp.exp(l - m)
    s = jnp.sum(p, axis=-1, keepdims=True)
    probs = p / s
    fg = probs[:, : C1 - 1]
    sc = jnp.max(fg, axis=-1, keepdims=True)
    sc_row = sc.T
    scores_ref[0] = sc_row

    cio = lax.broadcasted_iota(jnp.int32, (Q, C1 - 1), 1)
    cls = jnp.min(jnp.where(fg == sc, cio, C1), axis=-1, keepdims=True)

    ii = lax.broadcasted_iota(jnp.int32, (Q, Q), 0)
    jj = lax.broadcasted_iota(jnp.int32, (Q, Q), 1)
    beats = (sc_row > sc) | ((sc_row == sc) & (jj < ii))
    rank = jnp.sum(beats.astype(jnp.int32), axis=-1, keepdims=True)

    ro = lax.broadcasted_iota(jnp.int32, (Q, TOP_K), 1)
    onehot = rank == ro
    qio = lax.broadcasted_iota(jnp.int32, (Q, TOP_K), 0)
    tks_ref[0] = jnp.sum(jnp.where(onehot, sc, 0.0), axis=0, keepdims=True)
    tki_ref[0] = jnp.sum(jnp.where(onehot, qio, 0), axis=0, keepdims=True)
    tkc_ref[0] = jnp.sum(jnp.where(onehot, cls, 0), axis=0, keepdims=True)


def _resize_2x2(m):
    m4 = m.reshape(GH, 4, WM)
    rowsum = m4[:, 1, :] + m4[:, 2, :]
    rt4 = rowsum.T.reshape(GH, 4, GH)
    return (0.25 * (rt4[:, 1, :] + rt4[:, 2, :])).T


def _pool_body(idx_ref, *refs):
    mask_refs = refs[:TOP_K]
    v_ref = refs[TOP_K]
    out_ref = refs[TOP_K + 1]
    w_scratch = refs[TOP_K + 2]

    for k in range(TOP_K):
        resized = _resize_2x2(mask_refs[k][0, 0])
        w_scratch[:, k, :] = resized

    num = jnp.zeros((TOP_K, D), jnp.float32)
    den = jnp.zeros((TOP_K, 1), jnp.float32)
    for g in range(GH):
        sg = w_scratch[g]
        vg = v_ref[0, pl.ds(g * GH, GH), :]
        num = num + jnp.dot(sg, vg, preferred_element_type=jnp.float32)
        den = den + jnp.sum(sg, axis=1, keepdims=True)
    out_ref[0] = num / (den + EPS)


def _scores_topk(class_logits):
    return pl.pallas_call(
        _scores_topk_body,
        grid=(B,),
        in_specs=[pl.BlockSpec((1, Q, C1), lambda b: (b, 0, 0))],
        out_specs=(
            pl.BlockSpec((1, 1, Q), lambda b: (b, 0, 0)),
            pl.BlockSpec((1, 1, TOP_K), lambda b: (b, 0, 0)),
            pl.BlockSpec((1, 1, TOP_K), lambda b: (b, 0, 0)),
            pl.BlockSpec((1, 1, TOP_K), lambda b: (b, 0, 0)),
        ),
        out_shape=(
            jax.ShapeDtypeStruct((B, 1, Q), jnp.float32),
            jax.ShapeDtypeStruct((B, 1, TOP_K), jnp.float32),
            jax.ShapeDtypeStruct((B, 1, TOP_K), jnp.int32),
            jax.ShapeDtypeStruct((B, 1, TOP_K), jnp.int32),
        ),
    )(class_logits)


def _pool(topk_indices, soft_masks, visual_features):
    mask_specs = [
        pl.BlockSpec((1, 1, HM, WM), lambda b, idx, k=k: (b, idx[b, k], 0, 0))
        for k in range(TOP_K)
    ]
    grid_spec = pltpu.PrefetchScalarGridSpec(
        num_scalar_prefetch=1,
        grid=(B,),
        in_specs=mask_specs + [pl.BlockSpec((1, T, D), lambda b, idx: (b, 0, 0))],
        out_specs=pl.BlockSpec((1, TOP_K, D), lambda b, idx: (b, 0, 0)),
        scratch_shapes=[
            pltpu.VMEM((GH, TOP_K, GH), jnp.float32),
        ],
    )
    return pl.pallas_call(
        _pool_body,
        grid_spec=grid_spec,
        out_shape=jax.ShapeDtypeStruct((B, TOP_K, D), jnp.float32),
    )(topk_indices, *([soft_masks] * TOP_K), visual_features)


@functools.partial(jax.jit)
def kernel(soft_masks, visual_features, class_logits):
    scores, topk_scores, topk_indices, topk_class_ids = _scores_topk(class_logits)
    scores = scores.reshape(B, Q)
    topk_scores = topk_scores.reshape(B, TOP_K)
    topk_indices = topk_indices.reshape(B, TOP_K)
    topk_class_ids = topk_class_ids.reshape(B, TOP_K)
    pooled = _pool(topk_indices, soft_masks, visual_features)
    return pooled, topk_scores, topk_indices, topk_class_ids, scores


# speedup vs baseline: 4.1509x; 1.1708x over previous
import functools

import jax
import jax.numpy as jnp
from jax import lax
from jax.experimental import pallas as pl
from jax.experimental.pallas import tpu as pltpu

B, Q, HM, WM = 4, 200, 128, 128
T, D = 1024, 768
C1 = 81
TOP_K = 16
EPS = 1e-06
GH = 32


def _scores_topk_body(logits_ref, scores_ref, tks_ref, tki_ref, tkc_ref):
    l = logits_ref[...]
    m = jnp.max(l, axis=-1, keepdims=True)
    p = jnp.exp(l - m)
    s = jnp.sum(p, axis=-1, keepdims=True)
    probs = p / s
    fg = probs[:, :, : C1 - 1]
    sc3 = jnp.max(fg, axis=-1, keepdims=True)
    cio = lax.broadcasted_iota(jnp.int32, (B, Q, C1 - 1), 2)
    cls3 = jnp.min(jnp.where(fg == sc3, cio, C1), axis=-1, keepdims=True)

    ii = lax.broadcasted_iota(jnp.int32, (Q, Q), 0)
    jj = lax.broadcasted_iota(jnp.int32, (Q, Q), 1)
    ro = lax.broadcasted_iota(jnp.int32, (Q, TOP_K), 1)
    qio = lax.broadcasted_iota(jnp.int32, (Q, TOP_K), 0)

    for b in range(B):
        sc = sc3[b]
        sc_row = sc.T
        scores_ref[pl.ds(b, 1), :] = sc_row

        beats = (sc_row > sc) | ((sc_row == sc) & (jj < ii))
        rank = jnp.sum(beats.astype(jnp.int32), axis=-1, keepdims=True)

        onehot = rank == ro
        tks_ref[pl.ds(b, 1), :] = jnp.sum(jnp.where(onehot, sc, 0.0), axis=0, keepdims=True)
        tki_ref[pl.ds(b, 1), :] = jnp.sum(jnp.where(onehot, qio, 0), axis=0, keepdims=True)
        tkc_ref[pl.ds(b, 1), :] = jnp.sum(jnp.where(onehot, cls3[b], 0), axis=0, keepdims=True)


def _resize_2x2(m):
    m4 = m.reshape(GH, 4, WM)
    rowsum = m4[:, 1, :] + m4[:, 2, :]
    rt4 = rowsum.T.reshape(GH, 4, GH)
    return (0.25 * (rt4[:, 1, :] + rt4[:, 2, :])).T


def _pool_body(idx_ref, *refs):
    mask_refs = refs[:TOP_K]
    v_ref = refs[TOP_K]
    out_ref = refs[TOP_K + 1]
    w_scratch = refs[TOP_K + 2]

    for k in range(TOP_K):
        resized = _resize_2x2(mask_refs[k][0, 0])
        w_scratch[:, k, :] = resized

    num = jnp.zeros((TOP_K, D), jnp.float32)
    den = jnp.zeros((TOP_K, 1), jnp.float32)
    for g in range(GH):
        sg = w_scratch[g]
        vg = v_ref[0, pl.ds(g * GH, GH), :]
        num = num + jnp.dot(sg, vg, preferred_element_type=jnp.float32)
        den = den + jnp.sum(sg, axis=1, keepdims=True)
    out_ref[0] = num / (den + EPS)


def _scores_topk(class_logits):
    return pl.pallas_call(
        _scores_topk_body,
        out_shape=(
            jax.ShapeDtypeStruct((B, Q), jnp.float32),
            jax.ShapeDtypeStruct((B, TOP_K), jnp.float32),
            jax.ShapeDtypeStruct((B, TOP_K), jnp.int32),
            jax.ShapeDtypeStruct((B, TOP_K), jnp.int32),
        ),
    )(class_logits)


def _pool(topk_indices, soft_masks, visual_features):
    mask_specs = [
        pl.BlockSpec((1, 1, HM, WM), lambda b, idx, k=k: (b, idx[b, k], 0, 0))
        for k in range(TOP_K)
    ]
    grid_spec = pltpu.PrefetchScalarGridSpec(
        num_scalar_prefetch=1,
        grid=(B,),
        in_specs=mask_specs + [pl.BlockSpec((1, T, D), lambda b, idx: (b, 0, 0))],
        out_specs=pl.BlockSpec((1, TOP_K, D), lambda b, idx: (b, 0, 0)),
        scratch_shapes=[
            pltpu.VMEM((GH, TOP_K, GH), jnp.float32),
        ],
    )
    return pl.pallas_call(
        _pool_body,
        grid_spec=grid_spec,
        out_shape=jax.ShapeDtypeStruct((B, TOP_K, D), jnp.float32),
    )(topk_indices, *([soft_masks] * TOP_K), visual_features)


@functools.partial(jax.jit)
def kernel(soft_masks, visual_features, class_logits):
    scores, topk_scores, topk_indices, topk_class_ids = _scores_topk(class_logits)
    pooled = _pool(topk_indices, soft_masks, visual_features)
    return pooled, topk_scores, topk_indices, topk_class_ids, scores
